# baseline (speedup 1.0000x reference)
import jax
import jax.numpy as jnp
from jax.experimental import pallas as pl
from jax.experimental.pallas import tpu as pltpu

NEG = -3e38
ROWS = 8


def _shift_right(x, d, fill):
    pad = jnp.full(x.shape[:-1] + (d,), fill, x.dtype)
    return jnp.concatenate([pad, x[..., :-d]], axis=-1)


def _cumsum_j(x):
    for d in (1, 2, 4, 8, 16, 32, 64):
        x = x + _shift_right(x, d, 0.0)
    return x


def _cummax_j(x):
    for d in (1, 2, 4, 8, 16, 32, 64):
        x = jnp.maximum(x, _shift_right(x, d, NEG))
    return x


def _row_update(row, half_a, b):
    half_b = 0.5 * b
    S = _cumsum_j(_shift_right(half_b, 1, 0.0) + half_b)
    tmp = row + half_a
    cand_up = tmp + half_b
    cand_diag = _shift_right(tmp, 1, NEG) + half_b
    base = jnp.maximum(cand_up, cand_diag)
    return S + _cummax_j(base - S), half_b


def _dp_kernel(img_ref, out_ref, row_ref, prev_ref):
    R, Bb, J = img_ref.shape
    t = pl.program_id(1)

    @pl.when(t == 0)
    def _init():
        r0 = img_ref[0]
        half_r0 = 0.5 * r0
        S0 = _cumsum_j(_shift_right(half_r0, 1, 0.0) + half_r0)
        row = S0 + half_r0[:, 0:1]
        half_a = half_r0
        for r in range(1, R):
            row, half_a = _row_update(row, half_a, img_ref[r])
        row_ref[:, :] = row
        prev_ref[:, :] = half_a

    @pl.when(t != 0)
    def _step():
        row = row_ref[:, :]
        half_a = prev_ref[:, :]
        for r in range(R):
            row, half_a = _row_update(row, half_a, img_ref[r])
        row_ref[:, :] = row
        prev_ref[:, :] = half_a

    out_ref[:, :] = row_ref[:, J - 1 : J]


@jax.jit
def kernel(images):
    B, I, J = images.shape
    Bb = 128
    nb = B // Bb
    grid = (nb, I // ROWS)
    imgs_t = jnp.swapaxes(images, 0, 1)
    out = pl.pallas_call(
        _dp_kernel,
        grid=grid,
        in_specs=[pl.BlockSpec((ROWS, Bb, J), lambda b, t: (t, b, 0))],
        out_specs=pl.BlockSpec((Bb, 1), lambda b, t: (b, 0)),
        out_shape=jax.ShapeDtypeStruct((B, 1), jnp.float32),
        scratch_shapes=[
            pltpu.VMEM((Bb, J), jnp.float32),
            pltpu.VMEM((Bb, J), jnp.float32),
        ],
        compiler_params=pltpu.CompilerParams(
            dimension_semantics=("arbitrary", "arbitrary"),
        ),
    )(imgs_t)
    return out[:, 0]


# speedup vs baseline: 2.0237x; 2.0237x over previous
import jax
import jax.numpy as jnp
from jax.experimental import pallas as pl
from jax.experimental.pallas import tpu as pltpu

NEG = -3e38
ROWS = 8


def _shift_right(x, d, fill):
    rolled = jnp.roll(x, d, axis=-1)
    lane = jax.lax.broadcasted_iota(jnp.int32, x.shape, x.ndim - 1)
    return jnp.where(lane < d, fill, rolled)


def _cumsum_j(x):
    for d in (1, 2, 4, 8, 16, 32, 64):
        x = x + _shift_right(x, d, 0.0)
    return x


def _cummax_j(x):
    for d in (1, 2, 4, 8, 16, 32, 64):
        x = jnp.maximum(x, _shift_right(x, d, NEG))
    return x


def _row_update(row, half_a, b):
    half_b = 0.5 * b
    S = _cumsum_j(_shift_right(half_b, 1, 0.0) + half_b)
    tmp = row + half_a
    cand_up = tmp + half_b
    cand_diag = _shift_right(tmp, 1, NEG) + half_b
    base = jnp.maximum(cand_up, cand_diag)
    return S + _cummax_j(base - S), half_b


def _dp_kernel(img_ref, out_ref, row_ref, prev_ref):
    R, Bb, J = img_ref.shape
    t = pl.program_id(1)

    @pl.when(t == 0)
    def _init():
        r0 = img_ref[0]
        half_r0 = 0.5 * r0
        S0 = _cumsum_j(_shift_right(half_r0, 1, 0.0) + half_r0)
        row = S0 + half_r0[:, 0:1]
        half_a = half_r0
        for r in range(1, R):
            row, half_a = _row_update(row, half_a, img_ref[r])
        row_ref[:, :] = row
        prev_ref[:, :] = half_a

    @pl.when(t != 0)
    def _step():
        row = row_ref[:, :]
        half_a = prev_ref[:, :]
        for r in range(R):
            row, half_a = _row_update(row, half_a, img_ref[r])
        row_ref[:, :] = row
        prev_ref[:, :] = half_a

    out_ref[:, :] = row_ref[:, J - 1 : J]


@jax.jit
def kernel(images):
    B, I, J = images.shape
    Bb = 512
    nb = B // Bb
    grid = (nb, I // ROWS)
    imgs_t = jnp.swapaxes(images, 0, 1)
    out = pl.pallas_call(
        _dp_kernel,
        grid=grid,
        in_specs=[pl.BlockSpec((ROWS, Bb, J), lambda b, t: (t, b, 0))],
        out_specs=pl.BlockSpec((Bb, 1), lambda b, t: (b, 0)),
        out_shape=jax.ShapeDtypeStruct((B, 1), jnp.float32),
        scratch_shapes=[
            pltpu.VMEM((Bb, J), jnp.float32),
            pltpu.VMEM((Bb, J), jnp.float32),
        ],
        compiler_params=pltpu.CompilerParams(
            dimension_semantics=("arbitrary", "arbitrary"),
        ),
    )(imgs_t)
    return out[:, 0]


# speedup vs baseline: 3.2109x; 1.5867x over previous
import jax
import jax.numpy as jnp
from jax.experimental import pallas as pl
from jax.experimental.pallas import tpu as pltpu

NEG = -3e38
ROWS = 8


def _shift_right(x, d, fill):
    rolled = jnp.roll(x, d, axis=-1)
    lane = jax.lax.broadcasted_iota(jnp.int32, x.shape, x.ndim - 1)
    return jnp.where(lane < d, fill, rolled)


def _cumsum_j(x):
    for d in (1, 2, 4, 8, 16, 32, 64):
        x = x + _shift_right(x, d, 0.0)
    return x


def _cummax_j(x):
    for d in (1, 2, 4, 8, 16, 32, 64):
        x = jnp.maximum(x, _shift_right(x, d, NEG))
    return x


def _row_update(row, half_a, b, M):
    half_b = 0.5 * b
    S = jax.lax.dot_general(
        half_b, M, (((1,), (0,)), ((), ())),
        preferred_element_type=jnp.float32,
    )
    tmp = row + half_a
    cand_up = tmp + half_b
    cand_diag = _shift_right(tmp, 1, NEG) + half_b
    base = jnp.maximum(cand_up, cand_diag)
    return S + _cummax_j(base - S), half_b


def _dp_kernel(img_ref, m_ref, out_ref, row_ref, prev_ref):
    R, Bb, J = img_ref.shape
    t = pl.program_id(1)
    M = m_ref[:, :]

    @pl.when(t == 0)
    def _init():
        r0 = img_ref[0]
        half_r0 = 0.5 * r0
        S0 = jax.lax.dot_general(
            half_r0, M, (((1,), (0,)), ((), ())),
            preferred_element_type=jnp.float32,
        )
        row = S0 + (r0[:, 0:1] - S0[:, 0:1])
        half_a = half_r0
        for r in range(1, R):
            row, half_a = _row_update(row, half_a, img_ref[r], M)
        row_ref[:, :] = row
        prev_ref[:, :] = half_a

    @pl.when(t != 0)
    def _step():
        row = row_ref[:, :]
        half_a = prev_ref[:, :]
        for r in range(R):
            row, half_a = _row_update(row, half_a, img_ref[r], M)
        row_ref[:, :] = row
        prev_ref[:, :] = half_a

    out_ref[:, :] = row_ref[:, J - 1 : J]


@jax.jit
def kernel(images):
    B, I, J = images.shape
    Bb = 512
    nb = B // Bb
    grid = (nb, I // ROWS)
    imgs_t = jnp.swapaxes(images, 0, 1)
    k = jnp.arange(J)
    M = (2.0 * (k[:, None] < k[None, :]) + (k[:, None] == k[None, :])).astype(
        jnp.float32
    )
    out = pl.pallas_call(
        _dp_kernel,
        grid=grid,
        in_specs=[
            pl.BlockSpec((ROWS, Bb, J), lambda b, t: (t, b, 0)),
            pl.BlockSpec((J, J), lambda b, t: (0, 0)),
        ],
        out_specs=pl.BlockSpec((Bb, 1), lambda b, t: (b, 0)),
        out_shape=jax.ShapeDtypeStruct((B, 1), jnp.float32),
        scratch_shapes=[
            pltpu.VMEM((Bb, J), jnp.float32),
            pltpu.VMEM((Bb, J), jnp.float32),
        ],
        compiler_params=pltpu.CompilerParams(
            dimension_semantics=("arbitrary", "arbitrary"),
        ),
    )(imgs_t, M)
    return out[:, 0]
